# baseline (speedup 1.0000x reference)
import functools
import math

import jax
import jax.numpy as jnp
from jax.experimental import pallas as pl
from jax.experimental.pallas import tpu as pltpu


def _router_body(capacity, x_ref, cw_ref, mask_ref,
                 flat1_ref, w1_ref, flat2_ref, w2_ref):
    i = pl.program_id(0)
    T, E = x_ref.shape
    TB = cw_ref.shape[0]

    @pl.when(i == 0)
    def _compute_metadata():
        x = x_ref[...]
        m = jnp.max(x, axis=1, keepdims=True)
        ex = jnp.exp(x - m)
        p = ex / jnp.sum(ex, axis=1, keepdims=True)

        idx1 = jnp.argmax(p, axis=1).astype(jnp.int32)[:, None]
        p1 = jnp.max(p, axis=1, keepdims=True)
        eids = jax.lax.broadcasted_iota(jnp.int32, (T, E), 1)
        m1 = eids == idx1
        pm = jnp.where(m1, -jnp.inf, p)
        idx2 = jnp.argmax(pm, axis=1).astype(jnp.int32)[:, None]
        p2 = jnp.max(pm, axis=1, keepdims=True)
        m2 = eids == idx2

        def _cumsum0(v):
            s = 1
            while s < v.shape[0]:
                z = jnp.zeros((s, v.shape[1]), v.dtype)
                v = v + jnp.concatenate([z, v[:-s]], axis=0)
                s *= 2
            return v

        c1 = _cumsum0(m1.astype(jnp.int32))
        c2 = _cumsum0(m2.astype(jnp.int32))
        total1 = c1[T - 1:T, :]
        rank1 = jnp.sum(jnp.where(m1, c1, 0), axis=1, keepdims=True) - 1
        rank2 = jnp.sum(jnp.where(m2, c2 + total1, 0), axis=1, keepdims=True) - 1

        flat1 = jnp.where(rank1 < capacity, idx1 * capacity + rank1, -1)
        flat2 = jnp.where(rank2 < capacity, idx2 * capacity + rank2, -1)
        flat1_ref[...] = flat1
        flat2_ref[...] = flat2
        w1_ref[...] = p1
        w2_ref[...] = p2

    f1 = flat1_ref[pl.ds(i * TB, TB), :]
    f2 = flat2_ref[pl.ds(i * TB, TB), :]
    w1 = w1_ref[pl.ds(i * TB, TB), :]
    w2 = w2_ref[pl.ds(i * TB, TB), :]
    J = jax.lax.broadcasted_iota(jnp.int32, (TB, E * capacity), 1)
    out = jnp.where(J == f1, w1, 0.0)
    out = jnp.where(J == f2, w2, out)
    cw_ref[...] = out
    mask_ref[...] = out != 0.0


@jax.jit
def kernel(inputs):
    T, E = inputs.shape
    capacity = math.floor(2.0 * T / E)
    capacity += capacity % 2
    capacity = max(capacity, 4)

    TB = 256
    grid = (T // TB,)
    cw2d, mask2d = pl.pallas_call(
        functools.partial(_router_body, capacity),
        grid=grid,
        in_specs=[pl.BlockSpec((T, E), lambda i: (0, 0))],
        out_specs=[
            pl.BlockSpec((TB, E * capacity), lambda i: (i, 0)),
            pl.BlockSpec((TB, E * capacity), lambda i: (i, 0)),
        ],
        out_shape=[
            jax.ShapeDtypeStruct((T, E * capacity), jnp.float32),
            jax.ShapeDtypeStruct((T, E * capacity), jnp.bool_),
        ],
        scratch_shapes=[
            pltpu.VMEM((T, 1), jnp.int32),
            pltpu.VMEM((T, 1), jnp.float32),
            pltpu.VMEM((T, 1), jnp.int32),
            pltpu.VMEM((T, 1), jnp.float32),
        ],
    )(inputs.astype(jnp.float32))
    combine_weight = cw2d.reshape(T, E, capacity)
    sec_mask = mask2d.reshape(T, E, capacity)
    return combine_weight, sec_mask
